# baseline (speedup 1.0000x reference)
import jax
import jax.numpy as jnp
from jax.experimental import pallas as pl

DIM_C = 32
D = 64
SEQ = 65536
BLKC = 16384


def _contract(w, x):
    return jax.lax.dot_general(w, x, (((1,), (0,)), ((), ())),
                               preferred_element_type=jnp.float32)


def _body(z_ref, m_ref, fow_ref, bp_ref, out_ref):
    zb = z_ref[...]
    c = zb[:DIM_C, :]
    h = zb[DIM_C:, :]
    b12 = bp_ref[:, 0:1]
    b2 = bp_ref[:DIM_C, 1:2]
    a12 = _contract(m_ref[...], zb) + b12
    a1 = a12[:DIM_C, :]
    a2 = a12[DIM_C:, :]
    v1 = jnp.maximum(a1, 0.0) + (jnp.exp(jnp.minimum(a1, 0.0)) - 1.0)
    g = jnp.maximum(a2, 0.0) + jnp.log(1.0 + jnp.exp(-jnp.abs(a2)))
    v1p = jnp.concatenate([v1, jnp.zeros_like(v1)], axis=0)
    dc = _contract(fow_ref[...], v1p) + b2
    ts = jnp.concatenate([dc * c, c * c], axis=0)
    ri = jax.lax.broadcasted_iota(jnp.int32, (D, D), 0)
    rj = jax.lax.broadcasted_iota(jnp.int32, (D, D), 1)
    wred = ((ri < DIM_C) == (rj < DIM_C)).astype(jnp.float32)
    red = _contract(wred, ts)
    dcp = dc - (red[:DIM_C, :] / red[DIM_C:, :]) * c
    out_ref[...] = jnp.concatenate([dcp, -g * h], axis=0)


def kernel(t, z, F_cur_W, F_cur_b, F_out_W, F_out_b, G_W, G_b):
    m = jnp.concatenate(
        [F_cur_W, jnp.pad(G_W, ((0, 0), (0, D - DIM_C)))], axis=0)
    bp = jnp.stack([jnp.concatenate([F_cur_b, G_b]),
                    jnp.pad(F_out_b, (0, D - DIM_C))], axis=1)
    zt = jnp.transpose(z, (1, 2, 0)).reshape(D, SEQ)
    grid = (SEQ // BLKC,)
    full = lambda i: (0, 0)
    out = pl.pallas_call(
        _body,
        grid=grid,
        in_specs=[
            pl.BlockSpec((D, BLKC), lambda i: (0, i)),
            pl.BlockSpec((D, D), full),
            pl.BlockSpec((DIM_C, D), full),
            pl.BlockSpec((D, 2), full),
        ],
        out_specs=pl.BlockSpec((D, BLKC), lambda i: (0, i)),
        out_shape=jax.ShapeDtypeStruct((D, SEQ), jnp.float32),
    )(zt, m, F_out_W, bp)
    return jnp.transpose(out.reshape(1, D, SEQ), (2, 0, 1))


# speedup vs baseline: 1.1091x; 1.1091x over previous
import jax
import jax.numpy as jnp
from jax.experimental import pallas as pl

DIM_C = 32
D = 64
SEQ = 65536
BLKC = 16384


def _contract(w, x):
    return jax.lax.dot_general(w, x, (((1,), (0,)), ((), ())),
                               preferred_element_type=jnp.float32)


def _body(z_ref, fcw_ref, gw_ref, fow_ref, b3_ref, out_ref):
    zb = z_ref[...]
    c = zb[:DIM_C, :]
    h = zb[DIM_C:, :]
    b1 = b3_ref[:, 0:1]
    bg = b3_ref[:, 1:2]
    b2 = b3_ref[:, 2:3]
    a1 = _contract(fcw_ref[...], zb) + b1
    v1 = jnp.where(a1 > 0, a1, jnp.exp(jnp.minimum(a1, 0.0)) - 1.0)
    a2 = _contract(gw_ref[...], c) + bg
    g = jnp.maximum(a2, 0.0) + jnp.log(1.0 + jnp.exp(-jnp.abs(a2)))
    dc = _contract(fow_ref[:, :DIM_C], v1) + b2
    t = dc * c
    s = c * c
    ones = jnp.ones((DIM_C, DIM_C), jnp.float32)
    nb = _contract(ones, t)
    db = _contract(ones, s)
    dcp = dc - (nb / db) * c
    out_ref[...] = jnp.concatenate([dcp, -g * h], axis=0)


def kernel(t, z, F_cur_W, F_cur_b, F_out_W, F_out_b, G_W, G_b):
    b3 = jnp.stack([F_cur_b, G_b, F_out_b], axis=1)
    zt = jnp.transpose(z, (1, 2, 0)).reshape(D, SEQ)
    grid = (SEQ // BLKC,)
    full = lambda i: (0, 0)
    out = pl.pallas_call(
        _body,
        grid=grid,
        in_specs=[
            pl.BlockSpec((D, BLKC), lambda i: (0, i)),
            pl.BlockSpec((DIM_C, D), full),
            pl.BlockSpec((DIM_C, DIM_C), full),
            pl.BlockSpec((DIM_C, D), full),
            pl.BlockSpec((DIM_C, 3), full),
        ],
        out_specs=pl.BlockSpec((D, BLKC), lambda i: (0, i)),
        out_shape=jax.ShapeDtypeStruct((D, SEQ), jnp.float32),
    )(zt, F_cur_W, G_W, F_out_W, b3)
    return jnp.transpose(out.reshape(1, D, SEQ), (2, 0, 1))
